# Initial kernel scaffold; baseline (speedup 1.0000x reference)
import jax
import jax.numpy as jnp
from jax.experimental import pallas as pl


def kernel(edge_index, edge_attr, N):
    raise NotImplementedError("write your pallas kernel here")


# speedup vs baseline: 118.7778x; 118.7778x over previous
import functools

import jax
import jax.numpy as jnp
from jax import lax
from jax.experimental import pallas as pl
from jax.experimental.pallas import tpu as pltpu
from jax.experimental.pallas import tpu_sc as plsc

_N_NODES = 50000
_E = 3200000
_NC = 2
_NS = 16
_NW = _NC * _NS
_EPW = _E // _NW
_CH = 2000
_SLICE = 3136
_NPAD = _NS * _SLICE

_mesh = plsc.VectorSubcoreMesh(core_axis_name="c", subcore_axis_name="s")


@functools.partial(
    pl.kernel,
    mesh=_mesh,
    compiler_params=pltpu.CompilerParams(needs_layout_passes=False),
    out_type=jax.ShapeDtypeStruct((_NC * _NPAD,), jnp.float32),
    scratch_types=[
        pltpu.VMEM((_NPAD,), jnp.float32),
        pltpu.VMEM((_CH,), jnp.int32),
        pltpu.VMEM((_CH,), jnp.float32),
        pltpu.VMEM((_SLICE,), jnp.float32),
        pltpu.VMEM((_SLICE,), jnp.float32),
        pltpu.VMEM_SHARED((_NS * _NPAD,), jnp.float32),
    ],
)
def _hist_kernel(row_hbm, attr_hbm, out_hbm, hist, idxb, attrb, tmp, acc, shared):
    c = lax.axis_index("c")
    s = lax.axis_index("s")
    wid = c * _NS + s

    def zero_body(k, carry):
        hist[pl.ds(k * 16, 16)] = jnp.zeros((16,), jnp.float32)
        return carry
    lax.fori_loop(0, _NPAD // 16, zero_body, 0)

    base = wid * _EPW

    def chunk_body(j, carry):
        off = base + j * _CH
        pltpu.sync_copy(row_hbm.at[pl.ds(off, _CH)], idxb)
        pltpu.sync_copy(attr_hbm.at[pl.ds(off, _CH)], attrb)

        def vec_body(k, carry2):
            i = idxb[pl.ds(k * 16, 16)]
            a = attrb[pl.ds(k * 16, 16)]
            plsc.addupdate_scatter(hist, [i], a)
            return carry2
        lax.fori_loop(0, _CH // 16, vec_body, 0)
        return carry
    lax.fori_loop(0, _EPW // _CH, chunk_body, 0)

    pltpu.sync_copy(hist, shared.at[pl.ds(s * _NPAD, _NPAD)])
    plsc.subcore_barrier()

    soff = s * _SLICE
    pltpu.sync_copy(shared.at[pl.ds(soff, _SLICE)], acc)

    def red_body(t, carry):
        pltpu.sync_copy(shared.at[pl.ds(t * _NPAD + soff, _SLICE)], tmp)

        def add_body(k, carry2):
            acc[pl.ds(k * 16, 16)] = acc[pl.ds(k * 16, 16)] + tmp[pl.ds(k * 16, 16)]
            return carry2
        lax.fori_loop(0, _SLICE // 16, add_body, 0)
        return carry
    lax.fori_loop(1, _NS, red_body, 0)

    pltpu.sync_copy(acc, out_hbm.at[pl.ds(c * _NPAD + soff, _SLICE)])


@functools.partial(
    pl.kernel,
    mesh=_mesh,
    compiler_params=pltpu.CompilerParams(needs_layout_passes=False),
    out_type=jax.ShapeDtypeStruct((_E,), jnp.float32),
    scratch_types=[
        pltpu.VMEM((_NPAD,), jnp.float32),
        pltpu.VMEM((_CH,), jnp.int32),
        pltpu.VMEM((_CH,), jnp.float32),
        pltpu.VMEM((_CH,), jnp.float32),
    ],
)
def _norm_kernel(row_hbm, attr_hbm, norm_hbm, out_hbm, table, idxb, attrb, outb):
    c = lax.axis_index("c")
    s = lax.axis_index("s")
    wid = c * _NS + s

    pltpu.sync_copy(norm_hbm, table)
    base = wid * _EPW

    def chunk_body(j, carry):
        off = base + j * _CH
        pltpu.sync_copy(row_hbm.at[pl.ds(off, _CH)], idxb)
        pltpu.sync_copy(attr_hbm.at[pl.ds(off, _CH)], attrb)

        def vec_body(k, carry2):
            i = idxb[pl.ds(k * 16, 16)]
            a = attrb[pl.ds(k * 16, 16)]
            n = plsc.load_gather(table, [i])
            outb[pl.ds(k * 16, 16)] = a * n
            return carry2
        lax.fori_loop(0, _CH // 16, vec_body, 0)

        pltpu.sync_copy(outb, out_hbm.at[pl.ds(off, _CH)])
        return carry
    lax.fori_loop(0, _EPW // _CH, chunk_body, 0)


def kernel(edge_index, edge_attr, N):
    row = edge_index[0]
    partial = _hist_kernel(row, edge_attr).reshape(_NC, _NPAD)
    rownorm = 1.0 / (partial[0] + partial[1])
    return _norm_kernel(row, edge_attr, rownorm)


# speedup vs baseline: 153.3960x; 1.2915x over previous
import functools

import jax
import jax.numpy as jnp
from jax import lax
from jax.experimental import pallas as pl
from jax.experimental.pallas import tpu as pltpu
from jax.experimental.pallas import tpu_sc as plsc

_N_NODES = 50000
_E = 3200000
_NC = 2
_NS = 16
_NW = _NC * _NS
_EPW = _E // _NW
_CH = 2000
_SLICE = 3136
_NPAD = _NS * _SLICE

_mesh = plsc.VectorSubcoreMesh(core_axis_name="c", subcore_axis_name="s")


@functools.partial(
    pl.kernel,
    mesh=_mesh,
    compiler_params=pltpu.CompilerParams(needs_layout_passes=False),
    out_type=jax.ShapeDtypeStruct((_NC * _NPAD,), jnp.float32),
    scratch_types=[
        pltpu.VMEM((_NPAD,), jnp.float32),
        pltpu.VMEM((_CH,), jnp.int32),
        pltpu.VMEM((_CH,), jnp.float32),
        pltpu.VMEM((_SLICE,), jnp.float32),
        pltpu.VMEM((_SLICE,), jnp.float32),
        pltpu.VMEM_SHARED((_NS * _NPAD,), jnp.float32),
    ],
)
def _hist_kernel(row_hbm, attr_hbm, out_hbm, hist, idxb, attrb, tmp, acc, shared):
    c = lax.axis_index("c")
    s = lax.axis_index("s")
    wid = c * _NS + s

    @plsc.parallel_loop(0, _NPAD // 16, unroll=8)
    def zero_body(k):
        hist[pl.ds(k * 16, 16)] = jnp.zeros((16,), jnp.float32)

    base = wid * _EPW

    def chunk_body(j, carry):
        off = base + j * _CH
        pltpu.sync_copy(row_hbm.at[pl.ds(off, _CH)], idxb)
        pltpu.sync_copy(attr_hbm.at[pl.ds(off, _CH)], attrb)

        @plsc.parallel_loop(0, _CH // 16, unroll=8)
        def vec_body(k):
            i = idxb[pl.ds(k * 16, 16)]
            a = attrb[pl.ds(k * 16, 16)]
            plsc.addupdate_scatter(hist, [i], a)
        return carry
    lax.fori_loop(0, _EPW // _CH, chunk_body, 0)

    pltpu.sync_copy(hist, shared.at[pl.ds(s * _NPAD, _NPAD)])
    plsc.subcore_barrier()

    soff = s * _SLICE
    pltpu.sync_copy(shared.at[pl.ds(soff, _SLICE)], acc)

    def red_body(t, carry):
        pltpu.sync_copy(shared.at[pl.ds(t * _NPAD + soff, _SLICE)], tmp)

        @plsc.parallel_loop(0, _SLICE // 16, unroll=8)
        def add_body(k):
            acc[pl.ds(k * 16, 16)] = acc[pl.ds(k * 16, 16)] + tmp[pl.ds(k * 16, 16)]
        return carry
    lax.fori_loop(1, _NS, red_body, 0)

    pltpu.sync_copy(acc, out_hbm.at[pl.ds(c * _NPAD + soff, _SLICE)])


@functools.partial(
    pl.kernel,
    mesh=_mesh,
    compiler_params=pltpu.CompilerParams(needs_layout_passes=False),
    out_type=jax.ShapeDtypeStruct((_E,), jnp.float32),
    scratch_types=[
        pltpu.VMEM((_NPAD,), jnp.float32),
        pltpu.VMEM((_CH,), jnp.int32),
        pltpu.VMEM((_CH,), jnp.float32),
        pltpu.VMEM((_CH,), jnp.float32),
    ],
)
def _norm_kernel(row_hbm, attr_hbm, norm_hbm, out_hbm, table, idxb, attrb, outb):
    c = lax.axis_index("c")
    s = lax.axis_index("s")
    wid = c * _NS + s

    pltpu.sync_copy(norm_hbm, table)
    base = wid * _EPW

    def chunk_body(j, carry):
        off = base + j * _CH
        pltpu.sync_copy(row_hbm.at[pl.ds(off, _CH)], idxb)
        pltpu.sync_copy(attr_hbm.at[pl.ds(off, _CH)], attrb)

        @plsc.parallel_loop(0, _CH // 16, unroll=8)
        def vec_body(k):
            i = idxb[pl.ds(k * 16, 16)]
            a = attrb[pl.ds(k * 16, 16)]
            n = plsc.load_gather(table, [i])
            outb[pl.ds(k * 16, 16)] = a * n

        pltpu.sync_copy(outb, out_hbm.at[pl.ds(off, _CH)])
        return carry
    lax.fori_loop(0, _EPW // _CH, chunk_body, 0)


def kernel(edge_index, edge_attr, N):
    row = edge_index[0]
    partial = _hist_kernel(row, edge_attr).reshape(_NC, _NPAD)
    rownorm = 1.0 / (partial[0] + partial[1])
    return _norm_kernel(row, edge_attr, rownorm)


# speedup vs baseline: 305.4453x; 1.9912x over previous
import functools

import jax
import jax.numpy as jnp
from jax import lax
from jax.experimental import pallas as pl
from jax.experimental.pallas import tpu as pltpu
from jax.experimental.pallas import tpu_sc as plsc

_N_NODES = 50000
_E = 3200000
_NC = 2
_NS = 16
_NW = _NC * _NS
_EPW = _E // _NW
_CH = 2000
_NCHUNK = _EPW // _CH
_SLICE = 3136
_NPAD = _NS * _SLICE

_mesh = plsc.VectorSubcoreMesh(core_axis_name="c", subcore_axis_name="s")


@functools.partial(
    pl.kernel,
    mesh=_mesh,
    compiler_params=pltpu.CompilerParams(needs_layout_passes=False),
    out_type=jax.ShapeDtypeStruct((_NC * _NPAD,), jnp.float32),
    scratch_types=[
        pltpu.VMEM((_NPAD,), jnp.float32),
        pltpu.VMEM((_CH,), jnp.int32),
        pltpu.VMEM((_CH,), jnp.int32),
        pltpu.VMEM((_CH,), jnp.float32),
        pltpu.VMEM((_CH,), jnp.float32),
        pltpu.VMEM((_SLICE,), jnp.float32),
        pltpu.VMEM((_SLICE,), jnp.float32),
        pltpu.VMEM_SHARED((_NS * _NPAD,), jnp.float32),
        pltpu.SemaphoreType.DMA,
        pltpu.SemaphoreType.DMA,
    ],
)
def _hist_kernel(row_hbm, attr_hbm, out_hbm, hist, idx0, idx1, attr0, attr1,
                 tmp, acc, shared, sem0, sem1):
    c = lax.axis_index("c")
    s = lax.axis_index("s")
    wid = c * _NS + s
    base = wid * _EPW

    idxb = (idx0, idx1)
    attrb = (attr0, attr1)
    sems = (sem0, sem1)

    def issue_load(j, p):
        off = base + j * _CH
        pltpu.make_async_copy(row_hbm.at[pl.ds(off, _CH)], idxb[p], sems[p]).start()
        pltpu.make_async_copy(attr_hbm.at[pl.ds(off, _CH)], attrb[p], sems[p]).start()

    def wait_load(p):
        pltpu.make_async_copy(row_hbm.at[pl.ds(0, _CH)], idxb[p], sems[p]).wait()
        pltpu.make_async_copy(attr_hbm.at[pl.ds(0, _CH)], attrb[p], sems[p]).wait()

    def compute(p):
        @plsc.parallel_loop(0, _CH // 16, unroll=8)
        def vec_body(k):
            i = idxb[p][pl.ds(k * 16, 16)]
            a = attrb[p][pl.ds(k * 16, 16)]
            plsc.addupdate_scatter(hist, [i], a)

    issue_load(0, 0)
    issue_load(1, 1)

    @plsc.parallel_loop(0, _NPAD // 16, unroll=8)
    def zero_body(k):
        hist[pl.ds(k * 16, 16)] = jnp.zeros((16,), jnp.float32)

    def pair_body(jj, carry):
        for p in range(2):
            j = 2 * jj + p
            wait_load(p)
            compute(p)
            issue_load(j + 2, p)
        return carry
    lax.fori_loop(0, _NCHUNK // 2 - 1, pair_body, 0)

    for p in range(2):
        wait_load(p)
        compute(p)

    pltpu.sync_copy(hist, shared.at[pl.ds(s * _NPAD, _NPAD)])
    plsc.subcore_barrier()

    soff = s * _SLICE
    pltpu.sync_copy(shared.at[pl.ds(soff, _SLICE)], acc)

    def red_body(t, carry):
        pltpu.sync_copy(shared.at[pl.ds(t * _NPAD + soff, _SLICE)], tmp)

        @plsc.parallel_loop(0, _SLICE // 16, unroll=8)
        def add_body(k):
            acc[pl.ds(k * 16, 16)] = acc[pl.ds(k * 16, 16)] + tmp[pl.ds(k * 16, 16)]
        return carry
    lax.fori_loop(1, _NS, red_body, 0)

    pltpu.sync_copy(acc, out_hbm.at[pl.ds(c * _NPAD + soff, _SLICE)])


@functools.partial(
    pl.kernel,
    mesh=_mesh,
    compiler_params=pltpu.CompilerParams(needs_layout_passes=False),
    out_type=jax.ShapeDtypeStruct((_E,), jnp.float32),
    scratch_types=[
        pltpu.VMEM((_NPAD,), jnp.float32),
        pltpu.VMEM((_CH,), jnp.int32),
        pltpu.VMEM((_CH,), jnp.int32),
        pltpu.VMEM((_CH,), jnp.float32),
        pltpu.VMEM((_CH,), jnp.float32),
        pltpu.VMEM((_CH,), jnp.float32),
        pltpu.VMEM((_CH,), jnp.float32),
        pltpu.SemaphoreType.DMA,
        pltpu.SemaphoreType.DMA,
        pltpu.SemaphoreType.DMA,
        pltpu.SemaphoreType.DMA,
    ],
)
def _norm_kernel(row_hbm, attr_hbm, norm_hbm, out_hbm, table,
                 idx0, idx1, attr0, attr1, out0, out1,
                 lsem0, lsem1, ssem0, ssem1):
    c = lax.axis_index("c")
    s = lax.axis_index("s")
    wid = c * _NS + s
    base = wid * _EPW

    idxb = (idx0, idx1)
    attrb = (attr0, attr1)
    outb = (out0, out1)
    lsems = (lsem0, lsem1)
    ssems = (ssem0, ssem1)

    def issue_load(j, p):
        off = base + j * _CH
        pltpu.make_async_copy(row_hbm.at[pl.ds(off, _CH)], idxb[p], lsems[p]).start()
        pltpu.make_async_copy(attr_hbm.at[pl.ds(off, _CH)], attrb[p], lsems[p]).start()

    def wait_load(p):
        pltpu.make_async_copy(row_hbm.at[pl.ds(0, _CH)], idxb[p], lsems[p]).wait()
        pltpu.make_async_copy(attr_hbm.at[pl.ds(0, _CH)], attrb[p], lsems[p]).wait()

    def issue_store(j, p):
        off = base + j * _CH
        pltpu.make_async_copy(outb[p], out_hbm.at[pl.ds(off, _CH)], ssems[p]).start()

    def wait_store(p):
        pltpu.make_async_copy(outb[p], out_hbm.at[pl.ds(0, _CH)], ssems[p]).wait()

    def compute(p):
        @plsc.parallel_loop(0, _CH // 16, unroll=8)
        def vec_body(k):
            i = idxb[p][pl.ds(k * 16, 16)]
            a = attrb[p][pl.ds(k * 16, 16)]
            n = plsc.load_gather(table, [i])
            outb[p][pl.ds(k * 16, 16)] = a * n

    issue_load(0, 0)
    issue_load(1, 1)
    pltpu.sync_copy(norm_hbm, table)

    for p in range(2):
        wait_load(p)
        compute(p)
        issue_store(p, p)
        issue_load(p + 2, p)

    def pair_body(jj, carry):
        for p in range(2):
            j = 2 * jj + p
            wait_load(p)
            wait_store(p)
            compute(p)
            issue_store(j, p)
            issue_load(j + 2, p)
        return carry
    lax.fori_loop(1, _NCHUNK // 2 - 1, pair_body, 0)

    for p in range(2):
        j = _NCHUNK - 2 + p
        wait_load(p)
        wait_store(p)
        compute(p)
        issue_store(j, p)

    for p in range(2):
        wait_store(p)


def kernel(edge_index, edge_attr, N):
    row = edge_index[0]
    partial = _hist_kernel(row, edge_attr).reshape(_NC, _NPAD)
    rownorm = 1.0 / (partial[0] + partial[1])
    return _norm_kernel(row, edge_attr, rownorm)


# speedup vs baseline: 366.0994x; 1.1986x over previous
import functools

import jax
import jax.numpy as jnp
from jax import lax
from jax.experimental import pallas as pl
from jax.experimental.pallas import tpu as pltpu
from jax.experimental.pallas import tpu_sc as plsc

_N_NODES = 50000
_E = 3200000
_NC = 2
_NS = 16
_NW = _NC * _NS
_EPW = _E // _NW
_CH = 4000
_NCHUNK = _EPW // _CH
_SLICE = 3136
_NPAD = _NS * _SLICE

_mesh = plsc.VectorSubcoreMesh(core_axis_name="c", subcore_axis_name="s")


@functools.partial(
    pl.kernel,
    mesh=_mesh,
    compiler_params=pltpu.CompilerParams(needs_layout_passes=False),
    out_type=jax.ShapeDtypeStruct((_NC * _NPAD,), jnp.float32),
    scratch_types=[
        pltpu.VMEM((_NPAD,), jnp.float32),
        pltpu.VMEM((_CH,), jnp.int32),
        pltpu.VMEM((_CH,), jnp.int32),
        pltpu.VMEM((_CH,), jnp.float32),
        pltpu.VMEM((_CH,), jnp.float32),
        pltpu.VMEM((_SLICE,), jnp.float32),
        pltpu.VMEM((_SLICE,), jnp.float32),
        pltpu.VMEM_SHARED((_NS * _NPAD,), jnp.float32),
        pltpu.SemaphoreType.DMA,
        pltpu.SemaphoreType.DMA,
    ],
)
def _hist_kernel(row_hbm, attr_hbm, out_hbm, hist, idx0, idx1, attr0, attr1,
                 tmp, acc, shared, sem0, sem1):
    c = lax.axis_index("c")
    s = lax.axis_index("s")
    wid = c * _NS + s
    base = wid * _EPW

    idxb = (idx0, idx1)
    attrb = (attr0, attr1)
    sems = (sem0, sem1)

    def issue_load(j, p):
        off = base + j * _CH
        pltpu.make_async_copy(row_hbm.at[pl.ds(off, _CH)], idxb[p], sems[p]).start()
        pltpu.make_async_copy(attr_hbm.at[pl.ds(off, _CH)], attrb[p], sems[p]).start()

    def wait_load(p):
        pltpu.make_async_copy(row_hbm.at[pl.ds(0, _CH)], idxb[p], sems[p]).wait()
        pltpu.make_async_copy(attr_hbm.at[pl.ds(0, _CH)], attrb[p], sems[p]).wait()

    def compute(p):
        @plsc.parallel_loop(0, _CH // 16, unroll=8)
        def vec_body(k):
            i = idxb[p][pl.ds(k * 16, 16)]
            a = attrb[p][pl.ds(k * 16, 16)]
            plsc.addupdate_scatter(hist, [i], a)

    issue_load(0, 0)
    issue_load(1, 1)

    @plsc.parallel_loop(0, _NPAD // 16, unroll=8)
    def zero_body(k):
        hist[pl.ds(k * 16, 16)] = jnp.zeros((16,), jnp.float32)

    def pair_body(jj, carry):
        for p in range(2):
            j = 2 * jj + p
            wait_load(p)
            compute(p)
            issue_load(j + 2, p)
        return carry
    lax.fori_loop(0, _NCHUNK // 2 - 1, pair_body, 0)

    for p in range(2):
        wait_load(p)
        compute(p)

    pltpu.sync_copy(hist, shared.at[pl.ds(s * _NPAD, _NPAD)])
    plsc.subcore_barrier()

    soff = s * _SLICE
    pltpu.sync_copy(shared.at[pl.ds(soff, _SLICE)], acc)

    def red_body(t, carry):
        pltpu.sync_copy(shared.at[pl.ds(t * _NPAD + soff, _SLICE)], tmp)

        @plsc.parallel_loop(0, _SLICE // 16, unroll=8)
        def add_body(k):
            acc[pl.ds(k * 16, 16)] = acc[pl.ds(k * 16, 16)] + tmp[pl.ds(k * 16, 16)]
        return carry
    lax.fori_loop(1, _NS, red_body, 0)

    pltpu.sync_copy(acc, out_hbm.at[pl.ds(c * _NPAD + soff, _SLICE)])


@functools.partial(
    pl.kernel,
    mesh=_mesh,
    compiler_params=pltpu.CompilerParams(needs_layout_passes=False),
    out_type=jax.ShapeDtypeStruct((_E,), jnp.float32),
    scratch_types=[
        pltpu.VMEM((_NPAD,), jnp.float32),
        pltpu.VMEM((_CH,), jnp.int32),
        pltpu.VMEM((_CH,), jnp.int32),
        pltpu.VMEM((_CH,), jnp.float32),
        pltpu.VMEM((_CH,), jnp.float32),
        pltpu.VMEM((_CH,), jnp.float32),
        pltpu.VMEM((_CH,), jnp.float32),
        pltpu.SemaphoreType.DMA,
        pltpu.SemaphoreType.DMA,
        pltpu.SemaphoreType.DMA,
        pltpu.SemaphoreType.DMA,
    ],
)
def _norm_kernel(row_hbm, attr_hbm, norm_hbm, out_hbm, table,
                 idx0, idx1, attr0, attr1, out0, out1,
                 lsem0, lsem1, ssem0, ssem1):
    c = lax.axis_index("c")
    s = lax.axis_index("s")
    wid = c * _NS + s
    base = wid * _EPW

    idxb = (idx0, idx1)
    attrb = (attr0, attr1)
    outb = (out0, out1)
    lsems = (lsem0, lsem1)
    ssems = (ssem0, ssem1)

    def issue_load(j, p):
        off = base + j * _CH
        pltpu.make_async_copy(row_hbm.at[pl.ds(off, _CH)], idxb[p], lsems[p]).start()
        pltpu.make_async_copy(attr_hbm.at[pl.ds(off, _CH)], attrb[p], lsems[p]).start()

    def wait_load(p):
        pltpu.make_async_copy(row_hbm.at[pl.ds(0, _CH)], idxb[p], lsems[p]).wait()
        pltpu.make_async_copy(attr_hbm.at[pl.ds(0, _CH)], attrb[p], lsems[p]).wait()

    def issue_store(j, p):
        off = base + j * _CH
        pltpu.make_async_copy(outb[p], out_hbm.at[pl.ds(off, _CH)], ssems[p]).start()

    def wait_store(p):
        pltpu.make_async_copy(outb[p], out_hbm.at[pl.ds(0, _CH)], ssems[p]).wait()

    def compute(p):
        @plsc.parallel_loop(0, _CH // 16, unroll=8)
        def vec_body(k):
            i = idxb[p][pl.ds(k * 16, 16)]
            a = attrb[p][pl.ds(k * 16, 16)]
            n = plsc.load_gather(table, [i])
            outb[p][pl.ds(k * 16, 16)] = a * n

    issue_load(0, 0)
    issue_load(1, 1)
    pltpu.sync_copy(norm_hbm, table)

    for p in range(2):
        wait_load(p)
        compute(p)
        issue_store(p, p)
        issue_load(p + 2, p)

    def pair_body(jj, carry):
        for p in range(2):
            j = 2 * jj + p
            wait_load(p)
            wait_store(p)
            compute(p)
            issue_store(j, p)
            issue_load(j + 2, p)
        return carry
    lax.fori_loop(1, _NCHUNK // 2 - 1, pair_body, 0)

    for p in range(2):
        j = _NCHUNK - 2 + p
        wait_load(p)
        wait_store(p)
        compute(p)
        issue_store(j, p)

    for p in range(2):
        wait_store(p)


def kernel(edge_index, edge_attr, N):
    row = edge_index[0]
    partial = _hist_kernel(row, edge_attr).reshape(_NC, _NPAD)
    rownorm = 1.0 / (partial[0] + partial[1])
    return _norm_kernel(row, edge_attr, rownorm)
